# baseline (speedup 1.0000x reference)
import functools

import jax
import jax.numpy as jnp
from jax import lax
from jax.experimental import pallas as pl
from jax.experimental.pallas import tpu as pltpu
from jax.experimental.pallas import tpu_sc as plsc

BATCH = 16384
EMB = 128

_INFO = plsc.get_sparse_core_info()
_NC, _NS = _INFO.num_cores, _INFO.num_subcores
_NW = _NC * _NS
_CH = 128

_mesh = plsc.VectorSubcoreMesh(core_axis_name="c", subcore_axis_name="s")

_SPLITS = (8192, 8192)


def _make_sc(nrows):
    bpw = nrows // _NW
    nch = bpw // _CH

    @functools.partial(
        pl.kernel,
        mesh=_mesh,
        out_type=[jax.ShapeDtypeStruct((nrows, EMB), jnp.float32)] * 4,
        scratch_types=[
            pltpu.VMEM((nch, _CH), jnp.int32),
            pltpu.VMEM((nch, _CH), jnp.int32),
            pltpu.VMEM((4, _CH, EMB), jnp.float32),
            [pltpu.SemaphoreType.DMA] * 4,
            [pltpu.SemaphoreType.DMA] * 4,
        ],
    )
    def sc_gather4(uidx_hbm, midx_hbm, ug_t, mg_t, um_t, mm_t,
                   ug_o, mg_o, um_o, mm_o,
                   uvec, mvec, ring, gsems, ssems):
        wid = lax.axis_index("s") * _NC + lax.axis_index("c")
        base = wid * bpw
        for c in range(nch):
            pltpu.sync_copy(uidx_hbm.at[pl.ds(base + c * _CH, _CH)],
                            uvec.at[c])
            pltpu.sync_copy(midx_hbm.at[pl.ds(base + c * _CH, _CH)],
                            mvec.at[c])

        jobs = []
        for tab, ivec, out in ((ug_t, uvec, ug_o), (mg_t, mvec, mg_o),
                               (um_t, uvec, um_o), (mm_t, mvec, mm_o)):
            for c in range(nch):
                jobs.append((tab, ivec, out, c))

        def gather(j):
            tab, ivec, _, c = jobs[j]
            return pltpu.make_async_copy(tab.at[ivec.at[c]], ring.at[j % 4],
                                         gsems[j % 4])

        def store(j):
            _, _, out, c = jobs[j]
            return pltpu.make_async_copy(ring.at[j % 4],
                                         out.at[pl.ds(base + c * _CH, _CH)],
                                         ssems[j % 4])

        n = len(jobs)
        for j in range(n):
            if j >= 4:
                store(j - 4).wait()
            gather(j).start()
            if j >= 1:
                gather(j - 1).wait()
                store(j - 1).start()
        gather(n - 1).wait()
        store(n - 1).start()
        for j in range(max(0, n - 4), n):
            store(j).wait()

    return sc_gather4


def _nt(a, b):
    return lax.dot_general(a, b, (((1,), (1,)), ((), ())),
                           preferred_element_type=jnp.float32)


def _tc_body(ug, mg, um, mm, gmfwr, w0at, w0bt, b0c, w1t, b1c, w2t, b2c,
             w3t, b3c, fmwt, cconst, out_ref):
    h = jnp.maximum(_nt(w0at[...], um[...]) + _nt(w0bt[...], mm[...])
                    + b0c[...], 0.0)
    h = jnp.maximum(jnp.dot(w1t[...], h, preferred_element_type=jnp.float32)
                    + b1c[...], 0.0)
    h = jnp.maximum(jnp.dot(w2t[...], h, preferred_element_type=jnp.float32)
                    + b2c[...], 0.0)
    h = jnp.maximum(jnp.dot(w3t[...], h, preferred_element_type=jnp.float32)
                    + b3c[...], 0.0)
    m = jnp.dot(fmwt[...], h, preferred_element_type=jnp.float32)
    g = _nt(gmfwr[...], ug[...] * mg[...])
    out_ref[...] = (m + g + cconst[...])[0]


def _full(shape):
    return pl.BlockSpec(shape, lambda i: (0, 0))


def _row(shape):
    return pl.BlockSpec(shape, lambda i: (i, 0))


def _make_tc(nrows):
    bm = nrows if nrows <= 4096 else nrows // 2
    return pl.pallas_call(
        _tc_body,
        grid=(nrows // bm,),
        in_specs=[
            _row((bm, EMB)),
            _row((bm, EMB)),
            _row((bm, EMB)),
            _row((bm, EMB)),
            _full((1, EMB)),
            _full((64, EMB)),
            _full((64, EMB)),
            _full((64, 1)),
            _full((32, 64)),
            _full((32, 1)),
            _full((16, 32)),
            _full((16, 1)),
            _full((8, 16)),
            _full((8, 1)),
            _full((1, 8)),
            _full((1, 1)),
        ],
        out_specs=pl.BlockSpec((bm,), lambda i: (i,)),
        out_shape=jax.ShapeDtypeStruct((nrows,), jnp.float32),
    )


_SC_CALLS = {n: _make_sc(n) for n in set(_SPLITS)}
_TC_CALLS = {n: _make_tc(n) for n in set(_SPLITS)}


def kernel(X, user_emb_gmf, movie_emb_gmf, user_emb_mlp, movie_emb_mlp,
           gmf_w, gmf_b, final_mlp_w, final_mlp_b, final_w, final_b,
           mlp_w0, mlp_b0, mlp_w1, mlp_b1, mlp_w2, mlp_b2, mlp_w3, mlp_b3):
    user = X[:, 0]
    movie = X[:, 1]
    fw0 = final_w[0, 0]
    fw1 = final_w[1, 0]
    gmfwr = (gmf_w[:, 0] * fw0).reshape(1, EMB)
    fmwt = (final_mlp_w[:, 0] * fw1).reshape(1, 8)
    cconst = (final_b[0] + fw0 * gmf_b[0] + fw1 * final_mlp_b[0]).reshape(1, 1)
    wts = (gmfwr, mlp_w0[:EMB].T, mlp_w0[EMB:].T, mlp_b0.reshape(-1, 1),
           mlp_w1.T, mlp_b1.reshape(-1, 1), mlp_w2.T, mlp_b2.reshape(-1, 1),
           mlp_w3.T, mlp_b3.reshape(-1, 1), fmwt, cconst)
    outs = []
    off = 0
    for nrows in _SPLITS:
        sl = slice(off, off + nrows)
        off += nrows
        ug, mg, um, mm = _SC_CALLS[nrows](user[sl], movie[sl],
                                          user_emb_gmf, movie_emb_gmf,
                                          user_emb_mlp, movie_emb_mlp)
        outs.append(_TC_CALLS[nrows](ug, mg, um, mm, *wts))
    return jnp.concatenate(outs).reshape(BATCH, 1)


# speedup vs baseline: 1.0533x; 1.0533x over previous
import functools

import jax
import jax.numpy as jnp
from jax import lax
from jax.experimental import pallas as pl
from jax.experimental.pallas import tpu as pltpu
from jax.experimental.pallas import tpu_sc as plsc

BATCH = 16384
EMB = 128

_INFO = plsc.get_sparse_core_info()
_NC, _NS = _INFO.num_cores, _INFO.num_subcores
_NW = _NC * _NS
_CH = 128

_mesh = plsc.VectorSubcoreMesh(core_axis_name="c", subcore_axis_name="s")

_SPLITS = (12288, 4096)


def _make_sc(nrows):
    bpw = nrows // _NW
    nch = bpw // _CH

    @functools.partial(
        pl.kernel,
        mesh=_mesh,
        out_type=[jax.ShapeDtypeStruct((nrows, EMB), jnp.float32)] * 4,
        scratch_types=[
            pltpu.VMEM((nch, _CH), jnp.int32),
            pltpu.VMEM((nch, _CH), jnp.int32),
            pltpu.VMEM((6, _CH, EMB), jnp.float32),
            [pltpu.SemaphoreType.DMA] * 6,
            [pltpu.SemaphoreType.DMA] * 6,
            [pltpu.SemaphoreType.DMA] * 2,
        ],
    )
    def sc_gather4(uidx_hbm, midx_hbm, ug_t, mg_t, um_t, mm_t,
                   ug_o, mg_o, um_o, mm_o,
                   uvec, mvec, ring, gsems, ssems, isems):
        wid = lax.axis_index("s") * _NC + lax.axis_index("c")
        base = wid * bpw
        idx_copies = []
        for c in range(nch):
            for src_hbm, dst, s in ((uidx_hbm, uvec, 0), (midx_hbm, mvec, 1)):
                d = pltpu.make_async_copy(
                    src_hbm.at[pl.ds(base + c * _CH, _CH)], dst.at[c],
                    isems[s])
                d.start()
                idx_copies.append(d)
        for d in idx_copies:
            d.wait()

        jobs = []
        for tab, ivec, out in ((ug_t, uvec, ug_o), (mg_t, mvec, mg_o),
                               (um_t, uvec, um_o), (mm_t, mvec, mm_o)):
            for c in range(nch):
                jobs.append((tab, ivec, out, c))

        def gather(j):
            tab, ivec, _, c = jobs[j]
            return pltpu.make_async_copy(tab.at[ivec.at[c]], ring.at[j % 6],
                                         gsems[j % 6])

        def store(j):
            _, _, out, c = jobs[j]
            return pltpu.make_async_copy(ring.at[j % 6],
                                         out.at[pl.ds(base + c * _CH, _CH)],
                                         ssems[j % 6])

        n = len(jobs)
        for j in range(n):
            if j >= 6:
                store(j - 6).wait()
            gather(j).start()
            if j >= 1:
                gather(j - 1).wait()
                store(j - 1).start()
        gather(n - 1).wait()
        store(n - 1).start()
        for j in range(max(0, n - 6), n):
            store(j).wait()

    return sc_gather4


def _nt(a, b):
    return lax.dot_general(a, b, (((1,), (1,)), ((), ())),
                           preferred_element_type=jnp.float32)


def _tc_body(ug, mg, um, mm, gmfwr, w0at, w0bt, b0c, w1t, b1c, w2t, b2c,
             w3t, b3c, fmwt, cconst, out_ref):
    h = jnp.maximum(_nt(w0at[...], um[...]) + _nt(w0bt[...], mm[...])
                    + b0c[...], 0.0)
    h = jnp.maximum(jnp.dot(w1t[...], h, preferred_element_type=jnp.float32)
                    + b1c[...], 0.0)
    h = jnp.maximum(jnp.dot(w2t[...], h, preferred_element_type=jnp.float32)
                    + b2c[...], 0.0)
    h = jnp.maximum(jnp.dot(w3t[...], h, preferred_element_type=jnp.float32)
                    + b3c[...], 0.0)
    m = jnp.dot(fmwt[...], h, preferred_element_type=jnp.float32)
    g = _nt(gmfwr[...], ug[...] * mg[...])
    out_ref[...] = (m + g + cconst[...])[0]


def _full(shape):
    return pl.BlockSpec(shape, lambda i: (0, 0))


def _row(shape):
    return pl.BlockSpec(shape, lambda i: (i, 0))


def _make_tc(nrows):
    bm = nrows if nrows <= 4096 else nrows // 2
    return pl.pallas_call(
        _tc_body,
        grid=(nrows // bm,),
        in_specs=[
            _row((bm, EMB)),
            _row((bm, EMB)),
            _row((bm, EMB)),
            _row((bm, EMB)),
            _full((1, EMB)),
            _full((64, EMB)),
            _full((64, EMB)),
            _full((64, 1)),
            _full((32, 64)),
            _full((32, 1)),
            _full((16, 32)),
            _full((16, 1)),
            _full((8, 16)),
            _full((8, 1)),
            _full((1, 8)),
            _full((1, 1)),
        ],
        out_specs=pl.BlockSpec((bm,), lambda i: (i,)),
        out_shape=jax.ShapeDtypeStruct((nrows,), jnp.float32),
    )


_SC_CALLS = {n: _make_sc(n) for n in set(_SPLITS)}
_TC_CALLS = {n: _make_tc(n) for n in set(_SPLITS)}


def kernel(X, user_emb_gmf, movie_emb_gmf, user_emb_mlp, movie_emb_mlp,
           gmf_w, gmf_b, final_mlp_w, final_mlp_b, final_w, final_b,
           mlp_w0, mlp_b0, mlp_w1, mlp_b1, mlp_w2, mlp_b2, mlp_w3, mlp_b3):
    user = X[:, 0]
    movie = X[:, 1]
    fw0 = final_w[0, 0]
    fw1 = final_w[1, 0]
    gmfwr = (gmf_w[:, 0] * fw0).reshape(1, EMB)
    fmwt = (final_mlp_w[:, 0] * fw1).reshape(1, 8)
    cconst = (final_b[0] + fw0 * gmf_b[0] + fw1 * final_mlp_b[0]).reshape(1, 1)
    wts = (gmfwr, mlp_w0[:EMB].T, mlp_w0[EMB:].T, mlp_b0.reshape(-1, 1),
           mlp_w1.T, mlp_b1.reshape(-1, 1), mlp_w2.T, mlp_b2.reshape(-1, 1),
           mlp_w3.T, mlp_b3.reshape(-1, 1), fmwt, cconst)
    outs = []
    off = 0
    for nrows in _SPLITS:
        sl = slice(off, off + nrows)
        off += nrows
        ug, mg, um, mm = _SC_CALLS[nrows](user[sl], movie[sl],
                                          user_emb_gmf, movie_emb_gmf,
                                          user_emb_mlp, movie_emb_mlp)
        outs.append(_TC_CALLS[nrows](ug, mg, um, mm, *wts))
    return jnp.concatenate(outs).reshape(BATCH, 1)
